# baseline (speedup 1.0000x reference)
import functools

import jax
import jax.numpy as jnp
from jax import lax
from jax.experimental import pallas as pl
from jax.experimental.pallas import tpu as pltpu
from jax.experimental.pallas import tpu_sc as plsc

_L = 16
_NC = 2
_NS = 16
_NW = _NC * _NS
_CHUNK = 128


@functools.lru_cache(maxsize=None)
def _build(n, ncols):
    per_w = -(-n // (_NW * _CHUNK)) * _CHUNK
    n_chunks = per_w // _CHUNK
    assert n >= per_w and (n - per_w) % 8 == 0
    mesh = plsc.VectorSubcoreMesh(core_axis_name="c", subcore_axis_name="s")

    @functools.partial(
        pl.kernel,
        mesh=mesh,
        out_type=jax.ShapeDtypeStruct((n,), jnp.float32),
        scratch_types=[
            pltpu.VMEM((2 * per_w,), jnp.int32),
            pltpu.VMEM((n_chunks, _CHUNK), jnp.int32),
            pltpu.VMEM((per_w,), jnp.float32),
            pltpu.SemaphoreType.DMA,
        ],
        compiler_params=pltpu.CompilerParams(needs_layout_passes=False),
    )
    def gather_kernel(xflat, pairs_hbm, out_hbm, pairs_v, lin_v, vals_v, sem):
        wid = lax.axis_index("s") * _NC + lax.axis_index("c")
        base = pl.multiple_of(jnp.minimum(wid * per_w, n - per_w), 8)
        pltpu.sync_copy(pairs_hbm.at[pl.ds(2 * base, 2 * per_w)], pairs_v)
        lane2 = lax.iota(jnp.int32, _L) * 2
        for j in range(n_chunks):
            for i in range(_CHUNK // _L):
                e = lane2 + (j * _CHUNK + i * _L) * 2
                r = plsc.load_gather(pairs_v, [e])
                c = plsc.load_gather(pairs_v, [e + 1])
                lin_v[j, pl.ds(i * _L, _L)] = r * ncols + c
        copies = [
            pltpu.async_copy(xflat.at[lin_v.at[j]],
                             vals_v.at[pl.ds(j * _CHUNK, _CHUNK)], sem)
            for j in range(n_chunks)
        ]
        for cp in copies:
            cp.wait()
        pltpu.sync_copy(vals_v, out_hbm.at[pl.ds(base, per_w)])

    return gather_kernel


def kernel(X, indices):
    n = indices.shape[0]
    out = _build(n, X.shape[1])(X.reshape(-1), indices.reshape(-1))
    return out.reshape(-1, 2)


# speedup vs baseline: 2.0221x; 2.0221x over previous
import functools

import jax
import jax.numpy as jnp
from jax import lax
from jax.experimental import pallas as pl
from jax.experimental.pallas import tpu as pltpu
from jax.experimental.pallas import tpu_sc as plsc

_NC = 2
_NS = 16
_NW = _NC * _NS
_CHUNK = 128


@functools.lru_cache(maxsize=None)
def _build(n):
    per_w = -(-n // (_NW * _CHUNK)) * _CHUNK
    n_chunks = per_w // _CHUNK
    assert n >= per_w and (n - per_w) % 8 == 0
    mesh = plsc.VectorSubcoreMesh(core_axis_name="c", subcore_axis_name="s")

    @functools.partial(
        pl.kernel,
        mesh=mesh,
        out_type=jax.ShapeDtypeStruct((n,), jnp.float32),
        scratch_types=[
            pltpu.VMEM((n_chunks, _CHUNK), jnp.int32),
            pltpu.VMEM((per_w,), jnp.float32),
            pltpu.SemaphoreType.DMA,
            pltpu.SemaphoreType.DMA,
        ],
    )
    def gather_kernel(xflat, lin_hbm, out_hbm, lin_v, vals_v, sem, sem_idx):
        wid = lax.axis_index("s") * _NC + lax.axis_index("c")
        base = pl.multiple_of(jnp.minimum(wid * per_w, n - per_w), 8)
        loads = [
            pltpu.async_copy(lin_hbm.at[pl.ds(base + j * _CHUNK, _CHUNK)],
                             lin_v.at[j], sem_idx)
            for j in range(n_chunks)
        ]
        for ld in loads:
            ld.wait()
        copies = [
            pltpu.async_copy(xflat.at[lin_v.at[j]],
                             vals_v.at[pl.ds(j * _CHUNK, _CHUNK)], sem)
            for j in range(n_chunks)
        ]
        for cp in copies:
            cp.wait()
        pltpu.sync_copy(vals_v, out_hbm.at[pl.ds(base, per_w)])

    return gather_kernel


def kernel(X, indices):
    n = indices.shape[0]
    h = n // 2
    lin = indices[:, 0] * X.shape[1] + indices[:, 1]
    lin2 = jnp.concatenate([lin[0::2], lin[1::2]])
    out = _build(n)(X.reshape(-1), lin2)
    return jnp.stack([out[:h], out[h:]], axis=1)


# speedup vs baseline: 2.5093x; 1.2410x over previous
import functools

import jax
import jax.numpy as jnp
from jax import lax
from jax.experimental import pallas as pl
from jax.experimental.pallas import tpu as pltpu
from jax.experimental.pallas import tpu_sc as plsc

_L = 16
_NC = 2
_NS = 16
_NW = _NC * _NS
_CHUNK = 128


@functools.lru_cache(maxsize=None)
def _build(n):
    h = n // 2
    per_w = -(-n // (_NW * _CHUNK)) * _CHUNK
    half_w = per_w // 2
    n_chunks = per_w // _CHUNK
    assert n >= per_w and (h - half_w) % 8 == 0
    mesh = plsc.VectorSubcoreMesh(core_axis_name="c", subcore_axis_name="s")

    @functools.partial(
        pl.kernel,
        mesh=mesh,
        out_type=jax.ShapeDtypeStruct((n,), jnp.float32),
        scratch_types=[
            pltpu.VMEM((n_chunks, _CHUNK), jnp.int32),
            pltpu.VMEM((per_w,), jnp.float32),
            pltpu.VMEM((half_w,), jnp.float32),
            pltpu.VMEM((half_w,), jnp.float32),
            pltpu.SemaphoreType.DMA,
            pltpu.SemaphoreType.DMA,
        ],
        compiler_params=pltpu.CompilerParams(needs_layout_passes=False),
    )
    def gather_kernel(xflat, lin_hbm, out_hbm,
                      lin_v, vals_v, b_v, d_v, sem, sem_idx):
        wid = lax.axis_index("s") * _NC + lax.axis_index("c")
        base = pl.multiple_of(jnp.minimum(wid * half_w, h - half_w), 8)
        loads = [
            pltpu.async_copy(
                lin_hbm.at[pl.ds(2 * base + j * _CHUNK, _CHUNK)],
                lin_v.at[j], sem_idx)
            for j in range(n_chunks)
        ]
        for ld in loads:
            ld.wait()
        copies = [
            pltpu.async_copy(xflat.at[lin_v.at[j]],
                             vals_v.at[pl.ds(j * _CHUNK, _CHUNK)], sem)
            for j in range(n_chunks)
        ]
        for cp in copies:
            cp.wait()
        lane2 = lax.iota(jnp.int32, _L) * 2
        for g in range(half_w // _L):
            e = lane2 + 2 * _L * g
            b_v[pl.ds(g * _L, _L)] = plsc.load_gather(vals_v, [e])
            d_v[pl.ds(g * _L, _L)] = plsc.load_gather(vals_v, [e + 1])
        pltpu.sync_copy(b_v, out_hbm.at[pl.ds(base, half_w)])
        pltpu.sync_copy(d_v, out_hbm.at[pl.ds(h + base, half_w)])

    return gather_kernel


def kernel(X, indices):
    n = indices.shape[0]
    h = n // 2
    lin = indices[:, 0] * X.shape[1] + indices[:, 1]
    out = _build(n)(X.reshape(-1), lin)
    return jnp.stack([out[:h], out[h:]], axis=1)


# speedup vs baseline: 2.7019x; 1.0767x over previous
import functools

import jax
import jax.numpy as jnp
from jax import lax
from jax.experimental import pallas as pl
from jax.experimental.pallas import tpu as pltpu
from jax.experimental.pallas import tpu_sc as plsc

_L = 16
_NC = 2
_NS = 16
_NW = _NC * _NS
_CHUNK = 128


@functools.lru_cache(maxsize=None)
def _build(n):
    h = n // 2
    per_w = -(-n // (_NW * _CHUNK)) * _CHUNK
    half_w = per_w // 2
    n_chunks = per_w // _CHUNK
    assert n >= per_w and (h - half_w) % 8 == 0
    mesh = plsc.VectorSubcoreMesh(core_axis_name="c", subcore_axis_name="s")

    @functools.partial(
        pl.kernel,
        mesh=mesh,
        out_type=jax.ShapeDtypeStruct((n,), jnp.float32),
        scratch_types=[
            pltpu.VMEM((n_chunks, _CHUNK), jnp.int32),
            pltpu.VMEM((per_w,), jnp.float32),
            pltpu.VMEM((half_w,), jnp.float32),
            pltpu.VMEM((half_w,), jnp.float32),
            pltpu.SemaphoreType.DMA,
            pltpu.SemaphoreType.DMA,
        ],
        compiler_params=pltpu.CompilerParams(needs_layout_passes=False),
    )
    def gather_kernel(xflat, lin_hbm, out_hbm,
                      lin_v, vals_v, b_v, d_v, sem, sem_idx):
        wid = lax.axis_index("s") * _NC + lax.axis_index("c")
        base = pl.multiple_of(jnp.minimum(wid * half_w, h - half_w), 8)
        loads = [
            pltpu.async_copy(
                lin_hbm.at[pl.ds(2 * base + j * _CHUNK, _CHUNK)],
                lin_v.at[j], sem_idx)
            for j in range(n_chunks)
        ]
        for ld in loads:
            ld.wait()
        copies = [
            pltpu.async_copy(xflat.at[lin_v.at[j]],
                             vals_v.at[pl.ds(j * _CHUNK, _CHUNK)], sem)
            for j in range(n_chunks)
        ]
        for cp in copies:
            cp.wait()
        lane2 = lax.iota(jnp.int32, _L) * 2
        for g in range(half_w // _L):
            e = lane2 + 2 * _L * g
            b_v[pl.ds(g * _L, _L)] = plsc.load_gather(vals_v, [e])
            d_v[pl.ds(g * _L, _L)] = plsc.load_gather(vals_v, [e + 1])
        pltpu.sync_copy(b_v, out_hbm.at[pl.ds(base, half_w)])
        pltpu.sync_copy(d_v, out_hbm.at[pl.ds(h + base, half_w)])

    return gather_kernel


def kernel(X, indices):
    n = indices.shape[0]
    h = n // 2
    nr, nc = X.shape
    r, c = indices[:, 0], indices[:, 1]
    if nr % 8 == 0 and nc % 128 == 0:
        lin = (((r >> 3) * (nc // 128) + (c >> 7)) << 10) + \
              ((r & 7) << 7) + (c & 127)
        xflat = X.reshape(nr // 8, 8, nc // 128, 128)
        xflat = xflat.transpose(0, 2, 1, 3).reshape(-1)
    else:
        lin = r * nc + c
        xflat = X.reshape(-1)
    out = _build(n)(xflat, lin)
    return jnp.stack([out[:h], out[h:]], axis=1)
